# baseline (speedup 1.0000x reference)
import functools

import jax
import jax.numpy as jnp
from jax import lax
from jax.experimental import pallas as pl
from jax.experimental.pallas import tpu as pltpu
from jax.experimental.pallas import tpu_sc as plsc

_CHUNK = 128


@functools.lru_cache(maxsize=None)
def _build(B, V, D):
    info = plsc.get_sparse_core_info()
    NC = info.num_cores
    NW = NC * info.num_subcores
    b_per_w = B // NW
    n_elem = b_per_w * D
    n_chunks = b_per_w // _CHUNK
    assert B % NW == 0 and D % 32 == 0 and b_per_w % _CHUNK == 0

    @functools.partial(
        pl.kernel,
        out_type=jax.ShapeDtypeStruct((B * D // 2,), jnp.int32),
        mesh=plsc.VectorSubcoreMesh(core_axis_name="c", subcore_axis_name="s"),
        compiler_params=pltpu.CompilerParams(use_tc_tiling_on_sc=False),
        scratch_types=[
            pltpu.VMEM((n_chunks, _CHUNK), jnp.int32),
            pltpu.VMEM((b_per_w, D), jnp.float32),
            pltpu.VMEM((n_elem // 2,), jnp.int32),
            pltpu.SemaphoreType.DMA,
        ],
    )
    def emb(table_hbm, ids_hbm, out_hbm, idx_v, rows_v, out_v, sem):
        wid = lax.axis_index("s") * NC + lax.axis_index("c")
        pltpu.sync_copy(ids_hbm.at[pl.ds(wid * n_chunks, n_chunks)], idx_v)
        copies = [
            pltpu.async_copy(
                table_hbm.at[idx_v.at[g]],
                rows_v.at[pl.ds(g * _CHUNK, _CHUNK)],
                sem,
            )
            for g in range(n_chunks)
        ]
        for cp in copies:
            cp.wait()

        lanes = lax.iota(jnp.int32, 16)
        idx_e = (lanes * 2) & 15
        idx_o = idx_e | 1
        lo_half = lanes < 8
        dnums = lax.GatherDimensionNumbers(
            offset_dims=(), collapsed_slice_dims=(0,), start_index_map=(0,)
        )

        def vgather(v, idx):
            return lax.gather(
                v,
                idx[:, None],
                dnums,
                slice_sizes=(1,),
                mode=lax.GatherScatterMode.PROMISE_IN_BOUNDS,
            )

        def rtne(u):
            lsb = lax.shift_right_logical(u, 16) & 1
            return lax.shift_right_logical(u + 0x7FFF + lsb, 16)

        def body(r, carry):
            for h in range(D // 32):
                va = rows_v[r, pl.ds(h * 32, 16)]
                vb = rows_v[r, pl.ds(h * 32 + 16, 16)]
                ev = jnp.where(lo_half, vgather(va, idx_e), vgather(vb, idx_e))
                od = jnp.where(lo_half, vgather(va, idx_o), vgather(vb, idx_o))
                we = rtne(lax.bitcast_convert_type(ev, jnp.int32))
                wo = rtne(lax.bitcast_convert_type(od, jnp.int32))
                out_v[pl.ds(r * (D // 2) + h * 16, 16)] = we | lax.shift_left(wo, 16)
            return carry

        lax.fori_loop(0, b_per_w, body, 0)
        pltpu.sync_copy(out_v, out_hbm.at[pl.ds(wid * (n_elem // 2), n_elem // 2)])

    return emb


def kernel(input_ids, weight):
    B, = input_ids.shape
    V, D = weight.shape
    ids2d = input_ids.reshape(-1, _CHUNK)
    words = _build(B, V, D)(weight, ids2d)
    return lax.bitcast_convert_type(words, jnp.bfloat16).reshape(B, D)


# speedup vs baseline: 1.0323x; 1.0323x over previous
import functools

import jax
import jax.numpy as jnp
from jax import lax
from jax.experimental import pallas as pl
from jax.experimental.pallas import tpu as pltpu
from jax.experimental.pallas import tpu_sc as plsc

_CHUNK = 128


@functools.lru_cache(maxsize=None)
def _build(B, V, D):
    info = plsc.get_sparse_core_info()
    NC = info.num_cores
    NW = NC * info.num_subcores
    b_per_w = B // NW
    n_elem = b_per_w * D
    n_chunks = b_per_w // _CHUNK
    assert B % NW == 0 and b_per_w % _CHUNK == 0

    @functools.partial(
        pl.kernel,
        out_type=jax.ShapeDtypeStruct((B, D), jnp.float32),
        mesh=plsc.VectorSubcoreMesh(core_axis_name="c", subcore_axis_name="s"),
        compiler_params=pltpu.CompilerParams(use_tc_tiling_on_sc=False),
        scratch_types=[
            pltpu.VMEM((n_chunks, _CHUNK), jnp.int32),
            pltpu.VMEM((b_per_w, D), jnp.float32),
            pltpu.SemaphoreType.DMA,
        ],
    )
    def emb(table_hbm, ids_hbm, out_hbm, idx_v, rows_v, sem):
        wid = lax.axis_index("s") * NC + lax.axis_index("c")
        pltpu.sync_copy(ids_hbm.at[pl.ds(wid * n_chunks, n_chunks)], idx_v)
        copies = [
            pltpu.async_copy(
                table_hbm.at[idx_v.at[g]],
                rows_v.at[pl.ds(g * _CHUNK, _CHUNK)],
                sem,
            )
            for g in range(n_chunks)
        ]
        for cp in copies:
            cp.wait()
        pltpu.sync_copy(
            rows_v,
            out_hbm.at[pl.ds(wid * b_per_w, b_per_w)],
        )

    return emb


def kernel(input_ids, weight):
    B, = input_ids.shape
    V, D = weight.shape
    ids2d = input_ids.reshape(-1, _CHUNK)
    rows = _build(B, V, D)(weight, ids2d)
    return rows.astype(jnp.bfloat16)


# speedup vs baseline: 1.0584x; 1.0252x over previous
import functools

import jax
import jax.numpy as jnp
from jax import lax
from jax.experimental import pallas as pl
from jax.experimental.pallas import tpu as pltpu
from jax.experimental.pallas import tpu_sc as plsc


@functools.lru_cache(maxsize=None)
def _build(B, V, D):
    info = plsc.get_sparse_core_info()
    NC = info.num_cores
    NW = NC * info.num_subcores
    b_per_w = B // NW
    assert B % NW == 0

    @functools.partial(
        pl.kernel,
        out_type=jax.ShapeDtypeStruct((B, D), jnp.float32),
        mesh=plsc.VectorSubcoreMesh(core_axis_name="c", subcore_axis_name="s"),
        scratch_types=[
            pltpu.VMEM((b_per_w,), jnp.int32),
            pltpu.SemaphoreType.DMA,
        ],
    )
    def emb(table_hbm, ids_hbm, out_hbm, idx_v, sem):
        wid = lax.axis_index("s") * NC + lax.axis_index("c")
        base = wid * b_per_w
        pltpu.sync_copy(ids_hbm.at[pl.ds(base, b_per_w)], idx_v)

        GRP = 16

        def issue(g, c):
            v = idx_v[pl.ds(g * GRP, GRP)]
            for k in range(GRP):
                pltpu.async_copy(
                    table_hbm.at[v[k]],
                    out_hbm.at[base + g * GRP + k],
                    sem,
                )
            return c

        lax.fori_loop(0, b_per_w // GRP, issue, 0)

        def drain(j, c):
            pltpu.make_async_copy(
                table_hbm.at[0],
                out_hbm.at[base],
                sem,
            ).wait()
            return c

        lax.fori_loop(0, b_per_w, drain, 0)

    return emb


def kernel(input_ids, weight):
    B, = input_ids.shape
    V, D = weight.shape
    rows = _build(B, V, D)(weight, input_ids)
    return rows.astype(jnp.bfloat16)


# speedup vs baseline: 1.7567x; 1.6598x over previous
import functools

import jax
import jax.numpy as jnp
from jax import lax
from jax.experimental import pallas as pl
from jax.experimental.pallas import tpu as pltpu
from jax.experimental.pallas import tpu_sc as plsc


@functools.lru_cache(maxsize=None)
def _build(B, V, D):
    info = plsc.get_sparse_core_info()
    NC = info.num_cores
    NW = NC * info.num_subcores
    b_per_w = B // NW
    assert B % NW == 0

    @functools.partial(
        pl.kernel,
        out_type=jax.ShapeDtypeStruct((B, D), jnp.float32),
        mesh=plsc.VectorSubcoreMesh(core_axis_name="c", subcore_axis_name="s"),
        scratch_types=[
            pltpu.VMEM((b_per_w,), jnp.int32),
            pltpu.VMEM((b_per_w, D), jnp.float32),
            pltpu.SemaphoreType.DMA,
        ],
    )
    def emb(table_hbm, ids_hbm, out_hbm, idx_v, rows_v, sem):
        wid = lax.axis_index("s") * NC + lax.axis_index("c")
        base = wid * b_per_w
        pltpu.sync_copy(ids_hbm.at[pl.ds(base, b_per_w)], idx_v)

        GRP = 16

        def issue(g, c):
            v = idx_v[pl.ds(g * GRP, GRP)]
            for k in range(GRP):
                pltpu.async_copy(
                    table_hbm.at[v[k]],
                    rows_v.at[g * GRP + k],
                    sem,
                )
            return c

        lax.fori_loop(0, b_per_w // GRP, issue, 0)

        def drain(j, c):
            pltpu.make_async_copy(
                table_hbm.at[0],
                rows_v.at[0],
                sem,
            ).wait()
            return c

        lax.fori_loop(0, b_per_w, drain, 0)

        pltpu.sync_copy(rows_v, out_hbm.at[pl.ds(base, b_per_w)])

    return emb


def kernel(input_ids, weight):
    B, = input_ids.shape
    V, D = weight.shape
    rows = _build(B, V, D)(weight, input_ids)
    return rows.astype(jnp.bfloat16)
